# Initial kernel scaffold; baseline (speedup 1.0000x reference)
import jax
import jax.numpy as jnp
from jax.experimental import pallas as pl


def kernel(x, Wl1, Wr1, a1, b1, Wl2, Wr2, a2, b2, Wl3, Wr3, a3, b3, Wl4, Wr4, a4, b4, Wm1, bm1, Wm2, bm2, Wg, bg, geod):
    raise NotImplementedError("write your pallas kernel here")


# speedup vs baseline: 3.9695x; 3.9695x over previous
import functools

import jax
import jax.numpy as jnp
from jax import lax
from jax.experimental import pallas as pl
from jax.experimental.pallas import tpu as pltpu
from jax.experimental.pallas import tpu_sc as plsc

_K = 16
_NEG = 0.2
_ALPHA = 2.0
_NP = 10240
_BR = 256
_CT = 2048
_BIG = 2 ** 30


def _proj_body(x_ref, wl_ref, wr_ref, hl_ref, hr_ref):
    x = x_ref[...]
    hl_ref[...] = jnp.dot(x, wl_ref[...], preferred_element_type=jnp.float32)
    hr_ref[...] = jnp.dot(x, wr_ref[...], preferred_element_type=jnp.float32)


def _proj(x, wl, wr):
    n, d = x.shape
    o = wl.shape[1]
    br = 512
    return pl.pallas_call(
        _proj_body,
        grid=(pl.cdiv(n, br),),
        in_specs=[pl.BlockSpec((br, d), lambda i: (i, 0)),
                  pl.BlockSpec((d, o), lambda i: (0, 0)),
                  pl.BlockSpec((d, o), lambda i: (0, 0))],
        out_specs=[pl.BlockSpec((br, o), lambda i: (i, 0)),
                   pl.BlockSpec((br, o), lambda i: (i, 0))],
        out_shape=[jax.ShapeDtypeStruct((n, o), jnp.float32),
                   jax.ShapeDtypeStruct((n, o), jnp.float32)],
    )(x, wl, wr)


def _knn_body(n_real, xb_ref, sqr_ref, xall_ref, sqc_ref, idx_ref, strip_ref):
    ib = pl.program_id(0)
    br = xb_ref.shape[0]
    np_ = xall_ref.shape[0]
    nct = np_ // _CT
    xb = xb_ref[...]
    sqr = sqr_ref[...]
    inf = jnp.float32(jnp.inf)

    def build(c, _):
        xc = xall_ref[pl.ds(c * _CT, _CT), :]
        dot = lax.dot_general(xb, xc, (((1,), (1,)), ((), ())),
                              preferred_element_type=jnp.float32)
        sqc = sqc_ref[:, pl.ds(c * _CT, _CT)]
        d2 = (sqr + sqc) - 2.0 * dot
        col = lax.broadcasted_iota(jnp.int32, (br, _CT), 1) + c * _CT
        rowg = lax.broadcasted_iota(jnp.int32, (br, _CT), 0) + ib * br
        d2 = jnp.where(col == rowg, inf, d2)
        d2 = jnp.where(col >= n_real, inf, d2)
        strip_ref[:, pl.ds(c * _CT, _CT)] = d2
        return 0

    lax.fori_loop(0, nct, build, 0)

    def select_k(k, idx_acc):
        def scan(c, carry):
            m, mi = carry
            t = strip_ref[:, pl.ds(c * _CT, _CT)]
            col = lax.broadcasted_iota(jnp.int32, (br, _CT), 1) + c * _CT
            tm = jnp.min(t, axis=1, keepdims=True)
            ti = jnp.min(jnp.where(t == tm, col, _BIG), axis=1, keepdims=True)
            better = (tm < m) | ((tm == m) & (ti < mi))
            return (jnp.where(better, tm, m), jnp.where(better, ti, mi))

        m0 = jnp.full((br, 1), inf, jnp.float32)
        i0 = jnp.full((br, 1), _BIG, jnp.int32)
        _, mi = lax.fori_loop(0, nct, scan, (m0, i0))

        def mask(c, _):
            col = lax.broadcasted_iota(jnp.int32, (br, _CT), 1) + c * _CT
            t = strip_ref[:, pl.ds(c * _CT, _CT)]
            strip_ref[:, pl.ds(c * _CT, _CT)] = jnp.where(col == mi, inf, t)
            return 0

        lax.fori_loop(0, nct, mask, 0)
        lane = lax.broadcasted_iota(jnp.int32, (br, _K), 1)
        return idx_acc + jnp.where(lane == k, mi, 0)

    idx_ref[...] = lax.fori_loop(0, _K, select_k,
                                 jnp.zeros((br, _K), jnp.int32))


def _knn(x_pad, sq_pad, n_real):
    np_, d = x_pad.shape
    body = functools.partial(_knn_body, n_real)
    return pl.pallas_call(
        body,
        grid=(np_ // _BR,),
        in_specs=[pl.BlockSpec((_BR, d), lambda i: (i, 0)),
                  pl.BlockSpec((_BR, 1), lambda i: (i, 0)),
                  pl.BlockSpec((np_, d), lambda i: (0, 0)),
                  pl.BlockSpec((1, np_), lambda i: (0, 0))],
        out_specs=pl.BlockSpec((_BR, _K), lambda i: (i, 0)),
        out_shape=jax.ShapeDtypeStruct((np_, _K), jnp.int32),
        scratch_shapes=[pltpu.VMEM((_BR, np_), jnp.float32)],
    )(x_pad, sq_pad.reshape(np_, 1), x_pad, sq_pad.reshape(1, np_))


def _sc_gather(table, idx_flat):
    info = plsc.get_sparse_core_info()
    nw = info.num_cores * info.num_subcores
    b, d = idx_flat.shape[0], table.shape[1]
    ch = 128
    bpw = b // nw
    nch = bpw // ch
    mesh = plsc.VectorSubcoreMesh(core_axis_name="c", subcore_axis_name="s")
    nc = info.num_cores

    @functools.partial(
        pl.kernel, mesh=mesh,
        out_type=jax.ShapeDtypeStruct((b, d), jnp.float32),
        scratch_types=[pltpu.VMEM((ch,), jnp.int32),
                       pltpu.VMEM((ch, d), jnp.float32),
                       pltpu.SemaphoreType.DMA],
    )
    def gat(table_hbm, idx_hbm, out_hbm, idx_c, rows_v, sem):
        wid = lax.axis_index("s") * nc + lax.axis_index("c")
        base = wid * bpw

        def body(c, _):
            off = base + c * ch
            pltpu.sync_copy(idx_hbm.at[pl.ds(off, ch)], idx_c)
            pltpu.async_copy(table_hbm.at[idx_c], rows_v, sem).wait()
            pltpu.sync_copy(rows_v, out_hbm.at[pl.ds(off, ch)])
            return 0

        lax.fori_loop(0, nch, body, 0)

    return gat(table, idx_flat)


def _att_body(hl_ref, hrj_ref, att_ref, b_ref, out_ref):
    br = hl_ref.shape[0]
    def rbf(v):
        return v.astype(jnp.bfloat16).astype(jnp.float32)

    hl = hl_ref[...]
    att_b = rbf(att_ref[...])
    lane = lax.broadcasted_iota(jnp.int32, (br, _K), 1)
    logits = jnp.zeros((br, _K), jnp.float32)
    for k in range(_K):
        e = hl + hrj_ref[:, k, :]
        e = jnp.where(e > 0, e, _NEG * e)
        s = jnp.sum(rbf(e) * att_b, axis=1, keepdims=True)
        logits = logits + jnp.where(lane == k, s, 0.0)
    mx = jnp.max(logits, axis=1, keepdims=True)
    ex = jnp.exp(logits - mx)
    a = rbf(ex / jnp.sum(ex, axis=1, keepdims=True))
    acc = jnp.zeros_like(hl)
    for k in range(_K):
        acc = acc + a[:, k:k + 1] * rbf(hrj_ref[:, k, :])
    out_ref[...] = acc + b_ref[...]


def _att(hl, hrj3, att, b):
    n, o = hl.shape
    br = 256
    return pl.pallas_call(
        _att_body,
        grid=(pl.cdiv(n, br),),
        in_specs=[pl.BlockSpec((br, o), lambda i: (i, 0)),
                  pl.BlockSpec((br, _K, o), lambda i: (i, 0, 0)),
                  pl.BlockSpec((1, o), lambda i: (0, 0)),
                  pl.BlockSpec((1, o), lambda i: (0, 0))],
        out_specs=pl.BlockSpec((br, o), lambda i: (i, 0)),
        out_shape=jax.ShapeDtypeStruct((n, o), jnp.float32),
    )(hl, hrj3, att, b)


def _mlp_body(cat_ref, wm1_ref, bm1_ref, wm2_ref, bm2_ref, wg_ref, bg_ref,
              geod_ref, y_ref):
    m = jnp.dot(cat_ref[...], wm1_ref[...],
                preferred_element_type=jnp.float32) + bm1_ref[...]
    m = jnp.maximum(m, 0.0)
    m = jnp.dot(m, wm2_ref[...],
                preferred_element_type=jnp.float32) + bm2_ref[...]
    m = jnp.maximum(m, 0.0)
    y = jnp.dot(m, wg_ref[...],
                preferred_element_type=jnp.float32) + bg_ref[...]
    y_ref[...] = y * (1.0 - jnp.exp(-_ALPHA * geod_ref[...]))


def _mlp(cat, wm1, bm1, wm2, bm2, wg, bg, geod):
    n, dc = cat.shape
    br = 512
    return pl.pallas_call(
        _mlp_body,
        grid=(pl.cdiv(n, br),),
        in_specs=[pl.BlockSpec((br, dc), lambda i: (i, 0)),
                  pl.BlockSpec((dc, 256), lambda i: (0, 0)),
                  pl.BlockSpec((1, 256), lambda i: (0, 0)),
                  pl.BlockSpec((256, 64), lambda i: (0, 0)),
                  pl.BlockSpec((1, 64), lambda i: (0, 0)),
                  pl.BlockSpec((64, 3), lambda i: (0, 0)),
                  pl.BlockSpec((1, 3), lambda i: (0, 0)),
                  pl.BlockSpec((br, 1), lambda i: (i, 0))],
        out_specs=pl.BlockSpec((br, 3), lambda i: (i, 0)),
        out_shape=jax.ShapeDtypeStruct((n, 3), jnp.float32),
    )(cat, wm1, bm1.reshape(1, -1), wm2, bm2.reshape(1, -1), wg,
      bg.reshape(1, -1), geod)


def _layer(h, wl, wr, att, b):
    n = h.shape[0]
    o = wl.shape[1]
    hl, hr = _proj(h, wl, wr)
    sq = jnp.sum(h * h, axis=1)
    x_pad = jnp.pad(h, ((0, _NP - n), (0, 0)))
    sq_pad = jnp.pad(sq, (0, _NP - n))
    idx = _knn(x_pad, sq_pad, n)[:n]
    bflat = n * _K
    bpad = ((bflat + 4095) // 4096) * 4096
    idx_flat = jnp.pad(idx.reshape(-1), (0, bpad - bflat))
    hrj = _sc_gather(hr, idx_flat)[:bflat]
    hrj3 = hrj.reshape(n, _K, o)
    return _att(hl, hrj3, att.reshape(1, -1), b.reshape(1, -1))


def kernel(x, Wl1, Wr1, a1, b1, Wl2, Wr2, a2, b2, Wl3, Wr3, a3, b3,
           Wl4, Wr4, a4, b4, Wm1, bm1, Wm2, bm2, Wg, bg, geod):
    h1 = _layer(x, Wl1, Wr1, a1, b1)
    h2 = _layer(h1, Wl2, Wr2, a2, b2)
    h3 = _layer(h2, Wl3, Wr3, a3, b3)
    h4 = _layer(h3, Wl4, Wr4, a4, b4)
    cat = jnp.concatenate([x, h1, h2, h3, h4], axis=1)
    return _mlp(cat, Wm1, bm1, Wm2, bm2, Wg, bg, geod)


# speedup vs baseline: 4.0429x; 1.0185x over previous
import functools

import jax
import jax.numpy as jnp
from jax import lax
from jax.experimental import pallas as pl
from jax.experimental.pallas import tpu as pltpu
from jax.experimental.pallas import tpu_sc as plsc

_K = 16
_NEG = 0.2
_ALPHA = 2.0
_NP = 10240
_BR = 256
_CT = 2048
_BIG = 2 ** 30


def _proj_body(x_ref, wl_ref, wr_ref, hl_ref, hr_ref):
    x = x_ref[...]
    hl_ref[...] = jnp.dot(x, wl_ref[...], preferred_element_type=jnp.float32)
    hr_ref[...] = jnp.dot(x, wr_ref[...], preferred_element_type=jnp.float32)


def _proj(x, wl, wr):
    n, d = x.shape
    o = wl.shape[1]
    br = 512
    return pl.pallas_call(
        _proj_body,
        grid=(pl.cdiv(n, br),),
        in_specs=[pl.BlockSpec((br, d), lambda i: (i, 0)),
                  pl.BlockSpec((d, o), lambda i: (0, 0)),
                  pl.BlockSpec((d, o), lambda i: (0, 0))],
        out_specs=[pl.BlockSpec((br, o), lambda i: (i, 0)),
                   pl.BlockSpec((br, o), lambda i: (i, 0))],
        out_shape=[jax.ShapeDtypeStruct((n, o), jnp.float32),
                   jax.ShapeDtypeStruct((n, o), jnp.float32)],
    )(x, wl, wr)


def _knn_body(n_real, xb_ref, sqr_ref, xall_ref, sqc_ref, idx_ref, strip_ref):
    ib = pl.program_id(0)
    br = xb_ref.shape[0]
    np_ = xall_ref.shape[0]
    nct = np_ // _CT
    xb = xb_ref[...]
    sqr = sqr_ref[...]
    inf = jnp.float32(jnp.inf)

    def build(c, _):
        xc = xall_ref[pl.ds(c * _CT, _CT), :]
        dot = lax.dot_general(xb, xc, (((1,), (1,)), ((), ())),
                              preferred_element_type=jnp.float32)
        sqc = sqc_ref[:, pl.ds(c * _CT, _CT)]
        d2 = (sqr + sqc) - 2.0 * dot
        col = lax.broadcasted_iota(jnp.int32, (br, _CT), 1) + c * _CT
        rowg = lax.broadcasted_iota(jnp.int32, (br, _CT), 0) + ib * br
        d2 = jnp.where(col == rowg, inf, d2)
        d2 = jnp.where(col >= n_real, inf, d2)
        strip_ref[:, pl.ds(c * _CT, _CT)] = d2
        return 0

    lax.fori_loop(0, nct, build, 0)

    def select_k(k, carry):
        idx_acc, prev_mi = carry

        def scan(c, sc):
            m, mi = sc
            t = strip_ref[:, pl.ds(c * _CT, _CT)]
            col = lax.broadcasted_iota(jnp.int32, (br, _CT), 1) + c * _CT
            t = jnp.where(col == prev_mi, inf, t)
            strip_ref[:, pl.ds(c * _CT, _CT)] = t
            tm = jnp.min(t, axis=1, keepdims=True)
            ti = jnp.min(jnp.where(t == tm, col, _BIG), axis=1, keepdims=True)
            better = (tm < m) | ((tm == m) & (ti < mi))
            return (jnp.where(better, tm, m), jnp.where(better, ti, mi))

        m0 = jnp.full((br, 1), inf, jnp.float32)
        i0 = jnp.full((br, 1), _BIG, jnp.int32)
        _, mi = lax.fori_loop(0, nct, scan, (m0, i0))
        lane = lax.broadcasted_iota(jnp.int32, (br, _K), 1)
        return (idx_acc + jnp.where(lane == k, mi, 0), mi)

    idx0 = jnp.zeros((br, _K), jnp.int32)
    pm0 = jnp.full((br, 1), -1, jnp.int32)
    idx_ref[...] = lax.fori_loop(0, _K, select_k, (idx0, pm0))[0]


def _knn(x_pad, sq_pad, n_real):
    np_, d = x_pad.shape
    body = functools.partial(_knn_body, n_real)
    return pl.pallas_call(
        body,
        grid=(np_ // _BR,),
        in_specs=[pl.BlockSpec((_BR, d), lambda i: (i, 0)),
                  pl.BlockSpec((_BR, 1), lambda i: (i, 0)),
                  pl.BlockSpec((np_, d), lambda i: (0, 0)),
                  pl.BlockSpec((1, np_), lambda i: (0, 0))],
        out_specs=pl.BlockSpec((_BR, _K), lambda i: (i, 0)),
        out_shape=jax.ShapeDtypeStruct((np_, _K), jnp.int32),
        scratch_shapes=[pltpu.VMEM((_BR, np_), jnp.float32)],
    )(x_pad, sq_pad.reshape(np_, 1), x_pad, sq_pad.reshape(1, np_))


def _sc_gather(table, idx_flat):
    info = plsc.get_sparse_core_info()
    nw = info.num_cores * info.num_subcores
    b, d = idx_flat.shape[0], table.shape[1]
    ch = 128
    bpw = b // nw
    nch = bpw // ch
    mesh = plsc.VectorSubcoreMesh(core_axis_name="c", subcore_axis_name="s")
    nc = info.num_cores

    @functools.partial(
        pl.kernel, mesh=mesh,
        out_type=jax.ShapeDtypeStruct((b, d), jnp.float32),
        scratch_types=[pltpu.VMEM((ch,), jnp.int32),
                       pltpu.VMEM((ch, d), jnp.float32),
                       pltpu.SemaphoreType.DMA],
    )
    def gat(table_hbm, idx_hbm, out_hbm, idx_c, rows_v, sem):
        wid = lax.axis_index("s") * nc + lax.axis_index("c")
        base = wid * bpw

        def body(c, _):
            off = base + c * ch
            pltpu.sync_copy(idx_hbm.at[pl.ds(off, ch)], idx_c)
            pltpu.async_copy(table_hbm.at[idx_c], rows_v, sem).wait()
            pltpu.sync_copy(rows_v, out_hbm.at[pl.ds(off, ch)])
            return 0

        lax.fori_loop(0, nch, body, 0)

    return gat(table, idx_flat)


def _att_body(hl_ref, hrj_ref, att_ref, b_ref, out_ref):
    br = hl_ref.shape[0]
    def rbf(v):
        return v.astype(jnp.bfloat16).astype(jnp.float32)

    hl = hl_ref[...]
    att_b = rbf(att_ref[...])
    lane = lax.broadcasted_iota(jnp.int32, (br, _K), 1)
    logits = jnp.zeros((br, _K), jnp.float32)
    for k in range(_K):
        e = hl + hrj_ref[:, k, :]
        e = jnp.where(e > 0, e, _NEG * e)
        s = jnp.sum(rbf(e) * att_b, axis=1, keepdims=True)
        logits = logits + jnp.where(lane == k, s, 0.0)
    mx = jnp.max(logits, axis=1, keepdims=True)
    ex = jnp.exp(logits - mx)
    a = rbf(ex / jnp.sum(ex, axis=1, keepdims=True))
    acc = jnp.zeros_like(hl)
    for k in range(_K):
        acc = acc + a[:, k:k + 1] * rbf(hrj_ref[:, k, :])
    out_ref[...] = acc + b_ref[...]


def _att(hl, hrj3, att, b):
    n, o = hl.shape
    br = 256
    return pl.pallas_call(
        _att_body,
        grid=(pl.cdiv(n, br),),
        in_specs=[pl.BlockSpec((br, o), lambda i: (i, 0)),
                  pl.BlockSpec((br, _K, o), lambda i: (i, 0, 0)),
                  pl.BlockSpec((1, o), lambda i: (0, 0)),
                  pl.BlockSpec((1, o), lambda i: (0, 0))],
        out_specs=pl.BlockSpec((br, o), lambda i: (i, 0)),
        out_shape=jax.ShapeDtypeStruct((n, o), jnp.float32),
    )(hl, hrj3, att, b)


def _mlp_body(cat_ref, wm1_ref, bm1_ref, wm2_ref, bm2_ref, wg_ref, bg_ref,
              geod_ref, y_ref):
    m = jnp.dot(cat_ref[...], wm1_ref[...],
                preferred_element_type=jnp.float32) + bm1_ref[...]
    m = jnp.maximum(m, 0.0)
    m = jnp.dot(m, wm2_ref[...],
                preferred_element_type=jnp.float32) + bm2_ref[...]
    m = jnp.maximum(m, 0.0)
    y = jnp.dot(m, wg_ref[...],
                preferred_element_type=jnp.float32) + bg_ref[...]
    y_ref[...] = y * (1.0 - jnp.exp(-_ALPHA * geod_ref[...]))


def _mlp(cat, wm1, bm1, wm2, bm2, wg, bg, geod):
    n, dc = cat.shape
    br = 512
    return pl.pallas_call(
        _mlp_body,
        grid=(pl.cdiv(n, br),),
        in_specs=[pl.BlockSpec((br, dc), lambda i: (i, 0)),
                  pl.BlockSpec((dc, 256), lambda i: (0, 0)),
                  pl.BlockSpec((1, 256), lambda i: (0, 0)),
                  pl.BlockSpec((256, 64), lambda i: (0, 0)),
                  pl.BlockSpec((1, 64), lambda i: (0, 0)),
                  pl.BlockSpec((64, 3), lambda i: (0, 0)),
                  pl.BlockSpec((1, 3), lambda i: (0, 0)),
                  pl.BlockSpec((br, 1), lambda i: (i, 0))],
        out_specs=pl.BlockSpec((br, 3), lambda i: (i, 0)),
        out_shape=jax.ShapeDtypeStruct((n, 3), jnp.float32),
    )(cat, wm1, bm1.reshape(1, -1), wm2, bm2.reshape(1, -1), wg,
      bg.reshape(1, -1), geod)


def _layer(h, wl, wr, att, b):
    n = h.shape[0]
    o = wl.shape[1]
    hl, hr = _proj(h, wl, wr)
    sq = jnp.sum(h * h, axis=1)
    x_pad = jnp.pad(h, ((0, _NP - n), (0, 0)))
    sq_pad = jnp.pad(sq, (0, _NP - n))
    idx = _knn(x_pad, sq_pad, n)[:n]
    bflat = n * _K
    bpad = ((bflat + 4095) // 4096) * 4096
    idx_flat = jnp.pad(idx.reshape(-1), (0, bpad - bflat))
    hrj = _sc_gather(hr, idx_flat)[:bflat]
    hrj3 = hrj.reshape(n, _K, o)
    return _att(hl, hrj3, att.reshape(1, -1), b.reshape(1, -1))


def kernel(x, Wl1, Wr1, a1, b1, Wl2, Wr2, a2, b2, Wl3, Wr3, a3, b3,
           Wl4, Wr4, a4, b4, Wm1, bm1, Wm2, bm2, Wg, bg, geod):
    h1 = _layer(x, Wl1, Wr1, a1, b1)
    h2 = _layer(h1, Wl2, Wr2, a2, b2)
    h3 = _layer(h2, Wl3, Wr3, a3, b3)
    h4 = _layer(h3, Wl4, Wr4, a4, b4)
    cat = jnp.concatenate([x, h1, h2, h3, h4], axis=1)
    return _mlp(cat, Wm1, bm1, Wm2, bm2, Wg, bg, geod)


# speedup vs baseline: 4.3651x; 1.0797x over previous
import functools

import jax
import jax.numpy as jnp
from jax import lax
from jax.experimental import pallas as pl
from jax.experimental.pallas import tpu as pltpu
from jax.experimental.pallas import tpu_sc as plsc

_K = 16
_NEG = 0.2
_ALPHA = 2.0
_NP = 10240
_BR = 512
_CT = 2048
_BIG = 2 ** 30


def _proj_body(x_ref, wl_ref, wr_ref, hl_ref, hr_ref):
    x = x_ref[...]
    hl_ref[...] = jnp.dot(x, wl_ref[...], preferred_element_type=jnp.float32)
    hr_ref[...] = jnp.dot(x, wr_ref[...], preferred_element_type=jnp.float32)


def _proj(x, wl, wr):
    n, d = x.shape
    o = wl.shape[1]
    br = 512
    return pl.pallas_call(
        _proj_body,
        grid=(pl.cdiv(n, br),),
        in_specs=[pl.BlockSpec((br, d), lambda i: (i, 0)),
                  pl.BlockSpec((d, o), lambda i: (0, 0)),
                  pl.BlockSpec((d, o), lambda i: (0, 0))],
        out_specs=[pl.BlockSpec((br, o), lambda i: (i, 0)),
                   pl.BlockSpec((br, o), lambda i: (i, 0))],
        out_shape=[jax.ShapeDtypeStruct((n, o), jnp.float32),
                   jax.ShapeDtypeStruct((n, o), jnp.float32)],
    )(x, wl, wr)


def _knn_body(n_real, xb_ref, sqr_ref, xall_ref, sqc_ref, idx_ref, strip_ref):
    ib = pl.program_id(0)
    br = xb_ref.shape[0]
    np_ = xall_ref.shape[0]
    nct = np_ // _CT
    xb = xb_ref[...]
    sqr = sqr_ref[...]
    inf = jnp.float32(jnp.inf)

    def build(c, _):
        xc = xall_ref[pl.ds(c * _CT, _CT), :]
        dot = lax.dot_general(xb, xc, (((1,), (1,)), ((), ())),
                              preferred_element_type=jnp.float32)
        sqc = sqc_ref[:, pl.ds(c * _CT, _CT)]
        d2 = (sqr + sqc) - 2.0 * dot
        col = lax.broadcasted_iota(jnp.int32, (br, _CT), 1) + c * _CT
        rowg = lax.broadcasted_iota(jnp.int32, (br, _CT), 0) + ib * br
        d2 = jnp.where(col == rowg, inf, d2)
        d2 = jnp.where(col >= n_real, inf, d2)
        strip_ref[:, pl.ds(c * _CT, _CT)] = d2
        return 0

    lax.fori_loop(0, nct, build, 0)

    def select_k(k, carry):
        idx_acc, prev_mi = carry

        def scan(c, sc):
            m, mi = sc
            t = strip_ref[:, pl.ds(c * _CT, _CT)]
            col = lax.broadcasted_iota(jnp.int32, (br, _CT), 1) + c * _CT
            t = jnp.where(col == prev_mi, inf, t)
            strip_ref[:, pl.ds(c * _CT, _CT)] = t
            tm = jnp.min(t, axis=1, keepdims=True)
            ti = jnp.min(jnp.where(t == tm, col, _BIG), axis=1, keepdims=True)
            better = (tm < m) | ((tm == m) & (ti < mi))
            return (jnp.where(better, tm, m), jnp.where(better, ti, mi))

        m0 = jnp.full((br, 1), inf, jnp.float32)
        i0 = jnp.full((br, 1), _BIG, jnp.int32)
        _, mi = lax.fori_loop(0, nct, scan, (m0, i0))
        lane = lax.broadcasted_iota(jnp.int32, (br, _K), 1)
        return (idx_acc + jnp.where(lane == k, mi, 0), mi)

    idx0 = jnp.zeros((br, _K), jnp.int32)
    pm0 = jnp.full((br, 1), -1, jnp.int32)
    idx_ref[...] = lax.fori_loop(0, _K, select_k, (idx0, pm0))[0]


def _knn(x_pad, sq_pad, n_real):
    np_, d = x_pad.shape
    body = functools.partial(_knn_body, n_real)
    return pl.pallas_call(
        body,
        grid=(np_ // _BR,),
        in_specs=[pl.BlockSpec((_BR, d), lambda i: (i, 0)),
                  pl.BlockSpec((_BR, 1), lambda i: (i, 0)),
                  pl.BlockSpec((np_, d), lambda i: (0, 0)),
                  pl.BlockSpec((1, np_), lambda i: (0, 0))],
        out_specs=pl.BlockSpec((_BR, _K), lambda i: (i, 0)),
        out_shape=jax.ShapeDtypeStruct((np_, _K), jnp.int32),
        scratch_shapes=[pltpu.VMEM((_BR, np_), jnp.float32)],
    )(x_pad, sq_pad.reshape(np_, 1), x_pad, sq_pad.reshape(1, np_))


def _sc_gather(table, idx_flat):
    info = plsc.get_sparse_core_info()
    nw = info.num_cores * info.num_subcores
    b, d = idx_flat.shape[0], table.shape[1]
    ch = 128
    bpw = b // nw
    nch = bpw // ch
    mesh = plsc.VectorSubcoreMesh(core_axis_name="c", subcore_axis_name="s")
    nc = info.num_cores

    @functools.partial(
        pl.kernel, mesh=mesh,
        out_type=jax.ShapeDtypeStruct((b, d), jnp.float32),
        scratch_types=[pltpu.VMEM((ch,), jnp.int32),
                       pltpu.VMEM((ch, d), jnp.float32),
                       pltpu.SemaphoreType.DMA],
    )
    def gat(table_hbm, idx_hbm, out_hbm, idx_c, rows_v, sem):
        wid = lax.axis_index("s") * nc + lax.axis_index("c")
        base = wid * bpw

        def body(c, _):
            off = base + c * ch
            pltpu.sync_copy(idx_hbm.at[pl.ds(off, ch)], idx_c)
            pltpu.async_copy(table_hbm.at[idx_c], rows_v, sem).wait()
            pltpu.sync_copy(rows_v, out_hbm.at[pl.ds(off, ch)])
            return 0

        lax.fori_loop(0, nch, body, 0)

    return gat(table, idx_flat)


def _att_body(hl_ref, hrj_ref, att_ref, b_ref, out_ref):
    br = hl_ref.shape[0]
    def rbf(v):
        return v.astype(jnp.bfloat16).astype(jnp.float32)

    hl = hl_ref[...]
    att_b = rbf(att_ref[...])
    lane = lax.broadcasted_iota(jnp.int32, (br, _K), 1)
    logits = jnp.zeros((br, _K), jnp.float32)
    for k in range(_K):
        e = hl + hrj_ref[:, k, :]
        e = jnp.where(e > 0, e, _NEG * e)
        s = jnp.sum(rbf(e) * att_b, axis=1, keepdims=True)
        logits = logits + jnp.where(lane == k, s, 0.0)
    mx = jnp.max(logits, axis=1, keepdims=True)
    ex = jnp.exp(logits - mx)
    a = rbf(ex / jnp.sum(ex, axis=1, keepdims=True))
    acc = jnp.zeros_like(hl)
    for k in range(_K):
        acc = acc + a[:, k:k + 1] * rbf(hrj_ref[:, k, :])
    out_ref[...] = acc + b_ref[...]


def _att(hl, hrj3, att, b):
    n, o = hl.shape
    br = 256
    return pl.pallas_call(
        _att_body,
        grid=(pl.cdiv(n, br),),
        in_specs=[pl.BlockSpec((br, o), lambda i: (i, 0)),
                  pl.BlockSpec((br, _K, o), lambda i: (i, 0, 0)),
                  pl.BlockSpec((1, o), lambda i: (0, 0)),
                  pl.BlockSpec((1, o), lambda i: (0, 0))],
        out_specs=pl.BlockSpec((br, o), lambda i: (i, 0)),
        out_shape=jax.ShapeDtypeStruct((n, o), jnp.float32),
    )(hl, hrj3, att, b)


def _mlp_body(cat_ref, wm1_ref, bm1_ref, wm2_ref, bm2_ref, wg_ref, bg_ref,
              geod_ref, y_ref):
    m = jnp.dot(cat_ref[...], wm1_ref[...],
                preferred_element_type=jnp.float32) + bm1_ref[...]
    m = jnp.maximum(m, 0.0)
    m = jnp.dot(m, wm2_ref[...],
                preferred_element_type=jnp.float32) + bm2_ref[...]
    m = jnp.maximum(m, 0.0)
    y = jnp.dot(m, wg_ref[...],
                preferred_element_type=jnp.float32) + bg_ref[...]
    y_ref[...] = y * (1.0 - jnp.exp(-_ALPHA * geod_ref[...]))


def _mlp(cat, wm1, bm1, wm2, bm2, wg, bg, geod):
    n, dc = cat.shape
    br = 512
    return pl.pallas_call(
        _mlp_body,
        grid=(pl.cdiv(n, br),),
        in_specs=[pl.BlockSpec((br, dc), lambda i: (i, 0)),
                  pl.BlockSpec((dc, 256), lambda i: (0, 0)),
                  pl.BlockSpec((1, 256), lambda i: (0, 0)),
                  pl.BlockSpec((256, 64), lambda i: (0, 0)),
                  pl.BlockSpec((1, 64), lambda i: (0, 0)),
                  pl.BlockSpec((64, 3), lambda i: (0, 0)),
                  pl.BlockSpec((1, 3), lambda i: (0, 0)),
                  pl.BlockSpec((br, 1), lambda i: (i, 0))],
        out_specs=pl.BlockSpec((br, 3), lambda i: (i, 0)),
        out_shape=jax.ShapeDtypeStruct((n, 3), jnp.float32),
    )(cat, wm1, bm1.reshape(1, -1), wm2, bm2.reshape(1, -1), wg,
      bg.reshape(1, -1), geod)


def _layer(h, wl, wr, att, b):
    n = h.shape[0]
    o = wl.shape[1]
    hl, hr = _proj(h, wl, wr)
    sq = jnp.sum(h * h, axis=1)
    x_pad = jnp.pad(h, ((0, _NP - n), (0, 0)))
    sq_pad = jnp.pad(sq, (0, _NP - n))
    idx = _knn(x_pad, sq_pad, n)[:n]
    bflat = n * _K
    bpad = ((bflat + 4095) // 4096) * 4096
    idx_flat = jnp.pad(idx.reshape(-1), (0, bpad - bflat))
    hrj = _sc_gather(hr, idx_flat)[:bflat]
    hrj3 = hrj.reshape(n, _K, o)
    return _att(hl, hrj3, att.reshape(1, -1), b.reshape(1, -1))


def kernel(x, Wl1, Wr1, a1, b1, Wl2, Wr2, a2, b2, Wl3, Wr3, a3, b3,
           Wl4, Wr4, a4, b4, Wm1, bm1, Wm2, bm2, Wg, bg, geod):
    h1 = _layer(x, Wl1, Wr1, a1, b1)
    h2 = _layer(h1, Wl2, Wr2, a2, b2)
    h3 = _layer(h2, Wl3, Wr3, a3, b3)
    h4 = _layer(h3, Wl4, Wr4, a4, b4)
    cat = jnp.concatenate([x, h1, h2, h3, h4], axis=1)
    return _mlp(cat, Wm1, bm1, Wm2, bm2, Wg, bg, geod)


# speedup vs baseline: 4.3934x; 1.0065x over previous
import functools

import jax
import jax.numpy as jnp
from jax import lax
from jax.experimental import pallas as pl
from jax.experimental.pallas import tpu as pltpu
from jax.experimental.pallas import tpu_sc as plsc

_K = 16
_NEG = 0.2
_ALPHA = 2.0
_NP = 10240
_BR = 1024
_CT = 2048
_BIG = 2 ** 30


def _proj_body(x_ref, wl_ref, wr_ref, hl_ref, hr_ref):
    x = x_ref[...]
    hl_ref[...] = jnp.dot(x, wl_ref[...], preferred_element_type=jnp.float32)
    hr_ref[...] = jnp.dot(x, wr_ref[...], preferred_element_type=jnp.float32)


def _proj(x, wl, wr):
    n, d = x.shape
    o = wl.shape[1]
    br = 512
    return pl.pallas_call(
        _proj_body,
        grid=(pl.cdiv(n, br),),
        in_specs=[pl.BlockSpec((br, d), lambda i: (i, 0)),
                  pl.BlockSpec((d, o), lambda i: (0, 0)),
                  pl.BlockSpec((d, o), lambda i: (0, 0))],
        out_specs=[pl.BlockSpec((br, o), lambda i: (i, 0)),
                   pl.BlockSpec((br, o), lambda i: (i, 0))],
        out_shape=[jax.ShapeDtypeStruct((n, o), jnp.float32),
                   jax.ShapeDtypeStruct((n, o), jnp.float32)],
    )(x, wl, wr)


def _knn_body(n_real, xb_ref, sqr_ref, xall_ref, sqc_ref, idx_ref, strip_ref):
    ib = pl.program_id(0)
    br = xb_ref.shape[0]
    np_ = xall_ref.shape[0]
    nct = np_ // _CT
    xb = xb_ref[...]
    sqr = sqr_ref[...]
    inf = jnp.float32(jnp.inf)

    def build(c, _):
        xc = xall_ref[pl.ds(c * _CT, _CT), :]
        dot = lax.dot_general(xb, xc, (((1,), (1,)), ((), ())),
                              preferred_element_type=jnp.float32)
        sqc = sqc_ref[:, pl.ds(c * _CT, _CT)]
        d2 = (sqr + sqc) - 2.0 * dot
        col = lax.broadcasted_iota(jnp.int32, (br, _CT), 1) + c * _CT
        rowg = lax.broadcasted_iota(jnp.int32, (br, _CT), 0) + ib * br
        d2 = jnp.where(col == rowg, inf, d2)
        d2 = jnp.where(col >= n_real, inf, d2)
        strip_ref[:, pl.ds(c * _CT, _CT)] = d2
        return 0

    lax.fori_loop(0, nct, build, 0)

    def select_k(k, carry):
        idx_acc, prev_mi = carry

        def scan(c, sc):
            m, mi = sc
            t = strip_ref[:, pl.ds(c * _CT, _CT)]
            col = lax.broadcasted_iota(jnp.int32, (br, _CT), 1) + c * _CT
            t = jnp.where(col == prev_mi, inf, t)
            strip_ref[:, pl.ds(c * _CT, _CT)] = t
            tm = jnp.min(t, axis=1, keepdims=True)
            ti = jnp.min(jnp.where(t == tm, col, _BIG), axis=1, keepdims=True)
            better = (tm < m) | ((tm == m) & (ti < mi))
            return (jnp.where(better, tm, m), jnp.where(better, ti, mi))

        m0 = jnp.full((br, 1), inf, jnp.float32)
        i0 = jnp.full((br, 1), _BIG, jnp.int32)
        _, mi = lax.fori_loop(0, nct, scan, (m0, i0))
        lane = lax.broadcasted_iota(jnp.int32, (br, _K), 1)
        return (idx_acc + jnp.where(lane == k, mi, 0), mi)

    idx0 = jnp.zeros((br, _K), jnp.int32)
    pm0 = jnp.full((br, 1), -1, jnp.int32)
    idx_ref[...] = lax.fori_loop(0, _K, select_k, (idx0, pm0))[0]


def _knn(x_pad, sq_pad, n_real):
    np_, d = x_pad.shape
    body = functools.partial(_knn_body, n_real)
    return pl.pallas_call(
        body,
        grid=(np_ // _BR,),
        in_specs=[pl.BlockSpec((_BR, d), lambda i: (i, 0)),
                  pl.BlockSpec((_BR, 1), lambda i: (i, 0)),
                  pl.BlockSpec((np_, d), lambda i: (0, 0)),
                  pl.BlockSpec((1, np_), lambda i: (0, 0))],
        out_specs=pl.BlockSpec((_BR, _K), lambda i: (i, 0)),
        out_shape=jax.ShapeDtypeStruct((np_, _K), jnp.int32),
        scratch_shapes=[pltpu.VMEM((_BR, np_), jnp.float32)],
    )(x_pad, sq_pad.reshape(np_, 1), x_pad, sq_pad.reshape(1, np_))


def _sc_gather(table, idx_flat):
    info = plsc.get_sparse_core_info()
    nw = info.num_cores * info.num_subcores
    b, d = idx_flat.shape[0], table.shape[1]
    ch = 128
    bpw = b // nw
    nch = bpw // ch
    mesh = plsc.VectorSubcoreMesh(core_axis_name="c", subcore_axis_name="s")
    nc = info.num_cores

    @functools.partial(
        pl.kernel, mesh=mesh,
        out_type=jax.ShapeDtypeStruct((b, d), jnp.float32),
        scratch_types=[pltpu.VMEM((ch,), jnp.int32),
                       pltpu.VMEM((ch, d), jnp.float32),
                       pltpu.SemaphoreType.DMA],
    )
    def gat(table_hbm, idx_hbm, out_hbm, idx_c, rows_v, sem):
        wid = lax.axis_index("s") * nc + lax.axis_index("c")
        base = wid * bpw

        def body(c, _):
            off = base + c * ch
            pltpu.sync_copy(idx_hbm.at[pl.ds(off, ch)], idx_c)
            pltpu.async_copy(table_hbm.at[idx_c], rows_v, sem).wait()
            pltpu.sync_copy(rows_v, out_hbm.at[pl.ds(off, ch)])
            return 0

        lax.fori_loop(0, nch, body, 0)

    return gat(table, idx_flat)


def _att_body(hl_ref, hrj_ref, att_ref, b_ref, out_ref):
    br = hl_ref.shape[0]
    def rbf(v):
        return v.astype(jnp.bfloat16).astype(jnp.float32)

    hl = hl_ref[...]
    att_b = rbf(att_ref[...])
    lane = lax.broadcasted_iota(jnp.int32, (br, _K), 1)
    logits = jnp.zeros((br, _K), jnp.float32)
    for k in range(_K):
        e = hl + hrj_ref[:, k, :]
        e = jnp.where(e > 0, e, _NEG * e)
        s = jnp.sum(rbf(e) * att_b, axis=1, keepdims=True)
        logits = logits + jnp.where(lane == k, s, 0.0)
    mx = jnp.max(logits, axis=1, keepdims=True)
    ex = jnp.exp(logits - mx)
    a = rbf(ex / jnp.sum(ex, axis=1, keepdims=True))
    acc = jnp.zeros_like(hl)
    for k in range(_K):
        acc = acc + a[:, k:k + 1] * rbf(hrj_ref[:, k, :])
    out_ref[...] = acc + b_ref[...]


def _att(hl, hrj3, att, b):
    n, o = hl.shape
    br = 256
    return pl.pallas_call(
        _att_body,
        grid=(pl.cdiv(n, br),),
        in_specs=[pl.BlockSpec((br, o), lambda i: (i, 0)),
                  pl.BlockSpec((br, _K, o), lambda i: (i, 0, 0)),
                  pl.BlockSpec((1, o), lambda i: (0, 0)),
                  pl.BlockSpec((1, o), lambda i: (0, 0))],
        out_specs=pl.BlockSpec((br, o), lambda i: (i, 0)),
        out_shape=jax.ShapeDtypeStruct((n, o), jnp.float32),
    )(hl, hrj3, att, b)


def _mlp_body(cat_ref, wm1_ref, bm1_ref, wm2_ref, bm2_ref, wg_ref, bg_ref,
              geod_ref, y_ref):
    m = jnp.dot(cat_ref[...], wm1_ref[...],
                preferred_element_type=jnp.float32) + bm1_ref[...]
    m = jnp.maximum(m, 0.0)
    m = jnp.dot(m, wm2_ref[...],
                preferred_element_type=jnp.float32) + bm2_ref[...]
    m = jnp.maximum(m, 0.0)
    y = jnp.dot(m, wg_ref[...],
                preferred_element_type=jnp.float32) + bg_ref[...]
    y_ref[...] = y * (1.0 - jnp.exp(-_ALPHA * geod_ref[...]))


def _mlp(cat, wm1, bm1, wm2, bm2, wg, bg, geod):
    n, dc = cat.shape
    br = 512
    return pl.pallas_call(
        _mlp_body,
        grid=(pl.cdiv(n, br),),
        in_specs=[pl.BlockSpec((br, dc), lambda i: (i, 0)),
                  pl.BlockSpec((dc, 256), lambda i: (0, 0)),
                  pl.BlockSpec((1, 256), lambda i: (0, 0)),
                  pl.BlockSpec((256, 64), lambda i: (0, 0)),
                  pl.BlockSpec((1, 64), lambda i: (0, 0)),
                  pl.BlockSpec((64, 3), lambda i: (0, 0)),
                  pl.BlockSpec((1, 3), lambda i: (0, 0)),
                  pl.BlockSpec((br, 1), lambda i: (i, 0))],
        out_specs=pl.BlockSpec((br, 3), lambda i: (i, 0)),
        out_shape=jax.ShapeDtypeStruct((n, 3), jnp.float32),
    )(cat, wm1, bm1.reshape(1, -1), wm2, bm2.reshape(1, -1), wg,
      bg.reshape(1, -1), geod)


def _layer(h, wl, wr, att, b):
    n = h.shape[0]
    o = wl.shape[1]
    hl, hr = _proj(h, wl, wr)
    sq = jnp.sum(h * h, axis=1)
    x_pad = jnp.pad(h, ((0, _NP - n), (0, 0)))
    sq_pad = jnp.pad(sq, (0, _NP - n))
    idx = _knn(x_pad, sq_pad, n)[:n]
    bflat = n * _K
    bpad = ((bflat + 4095) // 4096) * 4096
    idx_flat = jnp.pad(idx.reshape(-1), (0, bpad - bflat))
    hrj = _sc_gather(hr, idx_flat)[:bflat]
    hrj3 = hrj.reshape(n, _K, o)
    return _att(hl, hrj3, att.reshape(1, -1), b.reshape(1, -1))


def kernel(x, Wl1, Wr1, a1, b1, Wl2, Wr2, a2, b2, Wl3, Wr3, a3, b3,
           Wl4, Wr4, a4, b4, Wm1, bm1, Wm2, bm2, Wg, bg, geod):
    h1 = _layer(x, Wl1, Wr1, a1, b1)
    h2 = _layer(h1, Wl2, Wr2, a2, b2)
    h3 = _layer(h2, Wl3, Wr3, a3, b3)
    h4 = _layer(h3, Wl4, Wr4, a4, b4)
    cat = jnp.concatenate([x, h1, h2, h3, h4], axis=1)
    return _mlp(cat, Wm1, bm1, Wm2, bm2, Wg, bg, geod)


# speedup vs baseline: 4.3993x; 1.0013x over previous
import functools

import jax
import jax.numpy as jnp
from jax import lax
from jax.experimental import pallas as pl
from jax.experimental.pallas import tpu as pltpu
from jax.experimental.pallas import tpu_sc as plsc

_K = 16
_NEG = 0.2
_ALPHA = 2.0
_NP = 10240
_BR = 512
_CT = 2048
_BIG = 2 ** 30


def _proj_body(x_ref, wl_ref, wr_ref, hl_ref, hr_ref):
    x = x_ref[...]
    hl_ref[...] = jnp.dot(x, wl_ref[...], preferred_element_type=jnp.float32)
    hr_ref[...] = jnp.dot(x, wr_ref[...], preferred_element_type=jnp.float32)


def _proj(x, wl, wr):
    n, d = x.shape
    o = wl.shape[1]
    br = 512
    return pl.pallas_call(
        _proj_body,
        grid=(pl.cdiv(n, br),),
        in_specs=[pl.BlockSpec((br, d), lambda i: (i, 0)),
                  pl.BlockSpec((d, o), lambda i: (0, 0)),
                  pl.BlockSpec((d, o), lambda i: (0, 0))],
        out_specs=[pl.BlockSpec((br, o), lambda i: (i, 0)),
                   pl.BlockSpec((br, o), lambda i: (i, 0))],
        out_shape=[jax.ShapeDtypeStruct((n, o), jnp.float32),
                   jax.ShapeDtypeStruct((n, o), jnp.float32)],
    )(x, wl, wr)


def _knn_body(n_real, xb_ref, sqr_ref, xall_ref, sqc_ref, idx_ref, strip_ref):
    ib = pl.program_id(0)
    br = xb_ref.shape[0]
    np_ = xall_ref.shape[0]
    nct = np_ // _CT
    xb = xb_ref[...]
    sqr = sqr_ref[...]
    inf = jnp.float32(jnp.inf)

    def build(c, carry):
        m, mi = carry
        xc = xall_ref[pl.ds(c * _CT, _CT), :]
        dot = lax.dot_general(xb, xc, (((1,), (1,)), ((), ())),
                              preferred_element_type=jnp.float32)
        sqc = sqc_ref[:, pl.ds(c * _CT, _CT)]
        d2 = (sqr + sqc) - 2.0 * dot
        col = lax.broadcasted_iota(jnp.int32, (br, _CT), 1) + c * _CT
        rowg = lax.broadcasted_iota(jnp.int32, (br, _CT), 0) + ib * br
        d2 = jnp.where(col == rowg, inf, d2)
        d2 = jnp.where(col >= n_real, inf, d2)
        strip_ref[:, pl.ds(c * _CT, _CT)] = d2
        tm = jnp.min(d2, axis=1, keepdims=True)
        ti = jnp.min(jnp.where(d2 == tm, col, _BIG), axis=1, keepdims=True)
        better = (tm < m) | ((tm == m) & (ti < mi))
        return (jnp.where(better, tm, m), jnp.where(better, ti, mi))

    bm0 = jnp.full((br, 1), inf, jnp.float32)
    bi0 = jnp.full((br, 1), _BIG, jnp.int32)
    _, mi0 = lax.fori_loop(0, nct, build, (bm0, bi0))

    def select_k(k, carry):
        idx_acc, prev_mi = carry

        def scan(c, sc):
            m, mi = sc
            t = strip_ref[:, pl.ds(c * _CT, _CT)]
            col = lax.broadcasted_iota(jnp.int32, (br, _CT), 1) + c * _CT
            t = jnp.where(col == prev_mi, inf, t)
            strip_ref[:, pl.ds(c * _CT, _CT)] = t
            tm = jnp.min(t, axis=1, keepdims=True)
            ti = jnp.min(jnp.where(t == tm, col, _BIG), axis=1, keepdims=True)
            better = (tm < m) | ((tm == m) & (ti < mi))
            return (jnp.where(better, tm, m), jnp.where(better, ti, mi))

        m0 = jnp.full((br, 1), inf, jnp.float32)
        i0 = jnp.full((br, 1), _BIG, jnp.int32)
        _, mi = lax.fori_loop(0, nct, scan, (m0, i0))
        lane = lax.broadcasted_iota(jnp.int32, (br, _K), 1)
        return (idx_acc + jnp.where(lane == k, mi, 0), mi)

    lane0 = lax.broadcasted_iota(jnp.int32, (br, _K), 1)
    idx0 = jnp.where(lane0 == 0, mi0, 0)
    idx_ref[...] = lax.fori_loop(1, _K, select_k, (idx0, mi0))[0]


def _knn(x_pad, sq_pad, n_real):
    np_, d = x_pad.shape
    body = functools.partial(_knn_body, n_real)
    return pl.pallas_call(
        body,
        grid=(np_ // _BR,),
        in_specs=[pl.BlockSpec((_BR, d), lambda i: (i, 0)),
                  pl.BlockSpec((_BR, 1), lambda i: (i, 0)),
                  pl.BlockSpec((np_, d), lambda i: (0, 0)),
                  pl.BlockSpec((1, np_), lambda i: (0, 0))],
        out_specs=pl.BlockSpec((_BR, _K), lambda i: (i, 0)),
        out_shape=jax.ShapeDtypeStruct((np_, _K), jnp.int32),
        scratch_shapes=[pltpu.VMEM((_BR, np_), jnp.float32)],
    )(x_pad, sq_pad.reshape(np_, 1), x_pad, sq_pad.reshape(1, np_))


def _sc_gather(table, idx_flat):
    info = plsc.get_sparse_core_info()
    nw = info.num_cores * info.num_subcores
    b, d = idx_flat.shape[0], table.shape[1]
    ch = 128
    bpw = b // nw
    nch = bpw // ch
    mesh = plsc.VectorSubcoreMesh(core_axis_name="c", subcore_axis_name="s")
    nc = info.num_cores

    @functools.partial(
        pl.kernel, mesh=mesh,
        out_type=jax.ShapeDtypeStruct((b, d), jnp.float32),
        scratch_types=[pltpu.VMEM((ch,), jnp.int32),
                       pltpu.VMEM((ch, d), jnp.float32),
                       pltpu.SemaphoreType.DMA],
    )
    def gat(table_hbm, idx_hbm, out_hbm, idx_c, rows_v, sem):
        wid = lax.axis_index("s") * nc + lax.axis_index("c")
        base = wid * bpw

        def body(c, _):
            off = base + c * ch
            pltpu.sync_copy(idx_hbm.at[pl.ds(off, ch)], idx_c)
            pltpu.async_copy(table_hbm.at[idx_c], rows_v, sem).wait()
            pltpu.sync_copy(rows_v, out_hbm.at[pl.ds(off, ch)])
            return 0

        lax.fori_loop(0, nch, body, 0)

    return gat(table, idx_flat)


def _att_body(hl_ref, hrj_ref, att_ref, b_ref, out_ref):
    br = hl_ref.shape[0]
    def rbf(v):
        return v.astype(jnp.bfloat16).astype(jnp.float32)

    hl = hl_ref[...]
    att_b = rbf(att_ref[...])
    lane = lax.broadcasted_iota(jnp.int32, (br, _K), 1)
    logits = jnp.zeros((br, _K), jnp.float32)
    for k in range(_K):
        e = hl + hrj_ref[:, k, :]
        e = jnp.where(e > 0, e, _NEG * e)
        s = jnp.sum(rbf(e) * att_b, axis=1, keepdims=True)
        logits = logits + jnp.where(lane == k, s, 0.0)
    mx = jnp.max(logits, axis=1, keepdims=True)
    ex = jnp.exp(logits - mx)
    a = rbf(ex / jnp.sum(ex, axis=1, keepdims=True))
    acc = jnp.zeros_like(hl)
    for k in range(_K):
        acc = acc + a[:, k:k + 1] * rbf(hrj_ref[:, k, :])
    out_ref[...] = acc + b_ref[...]


def _att(hl, hrj3, att, b):
    n, o = hl.shape
    br = 256
    return pl.pallas_call(
        _att_body,
        grid=(pl.cdiv(n, br),),
        in_specs=[pl.BlockSpec((br, o), lambda i: (i, 0)),
                  pl.BlockSpec((br, _K, o), lambda i: (i, 0, 0)),
                  pl.BlockSpec((1, o), lambda i: (0, 0)),
                  pl.BlockSpec((1, o), lambda i: (0, 0))],
        out_specs=pl.BlockSpec((br, o), lambda i: (i, 0)),
        out_shape=jax.ShapeDtypeStruct((n, o), jnp.float32),
    )(hl, hrj3, att, b)


def _mlp_body(cat_ref, wm1_ref, bm1_ref, wm2_ref, bm2_ref, wg_ref, bg_ref,
              geod_ref, y_ref):
    m = jnp.dot(cat_ref[...], wm1_ref[...],
                preferred_element_type=jnp.float32) + bm1_ref[...]
    m = jnp.maximum(m, 0.0)
    m = jnp.dot(m, wm2_ref[...],
                preferred_element_type=jnp.float32) + bm2_ref[...]
    m = jnp.maximum(m, 0.0)
    y = jnp.dot(m, wg_ref[...],
                preferred_element_type=jnp.float32) + bg_ref[...]
    y_ref[...] = y * (1.0 - jnp.exp(-_ALPHA * geod_ref[...]))


def _mlp(cat, wm1, bm1, wm2, bm2, wg, bg, geod):
    n, dc = cat.shape
    br = 512
    return pl.pallas_call(
        _mlp_body,
        grid=(pl.cdiv(n, br),),
        in_specs=[pl.BlockSpec((br, dc), lambda i: (i, 0)),
                  pl.BlockSpec((dc, 256), lambda i: (0, 0)),
                  pl.BlockSpec((1, 256), lambda i: (0, 0)),
                  pl.BlockSpec((256, 64), lambda i: (0, 0)),
                  pl.BlockSpec((1, 64), lambda i: (0, 0)),
                  pl.BlockSpec((64, 3), lambda i: (0, 0)),
                  pl.BlockSpec((1, 3), lambda i: (0, 0)),
                  pl.BlockSpec((br, 1), lambda i: (i, 0))],
        out_specs=pl.BlockSpec((br, 3), lambda i: (i, 0)),
        out_shape=jax.ShapeDtypeStruct((n, 3), jnp.float32),
    )(cat, wm1, bm1.reshape(1, -1), wm2, bm2.reshape(1, -1), wg,
      bg.reshape(1, -1), geod)


def _layer(h, wl, wr, att, b):
    n = h.shape[0]
    o = wl.shape[1]
    hl, hr = _proj(h, wl, wr)
    sq = jnp.sum(h * h, axis=1)
    x_pad = jnp.pad(h, ((0, _NP - n), (0, 0)))
    sq_pad = jnp.pad(sq, (0, _NP - n))
    idx = _knn(x_pad, sq_pad, n)[:n]
    bflat = n * _K
    bpad = ((bflat + 4095) // 4096) * 4096
    idx_flat = jnp.pad(idx.reshape(-1), (0, bpad - bflat))
    hrj = _sc_gather(hr, idx_flat)[:bflat]
    hrj3 = hrj.reshape(n, _K, o)
    return _att(hl, hrj3, att.reshape(1, -1), b.reshape(1, -1))


def kernel(x, Wl1, Wr1, a1, b1, Wl2, Wr2, a2, b2, Wl3, Wr3, a3, b3,
           Wl4, Wr4, a4, b4, Wm1, bm1, Wm2, bm2, Wg, bg, geod):
    h1 = _layer(x, Wl1, Wr1, a1, b1)
    h2 = _layer(h1, Wl2, Wr2, a2, b2)
    h3 = _layer(h2, Wl3, Wr3, a3, b3)
    h4 = _layer(h3, Wl4, Wr4, a4, b4)
    cat = jnp.concatenate([x, h1, h2, h3, h4], axis=1)
    return _mlp(cat, Wm1, bm1, Wm2, bm2, Wg, bg, geod)
